# baseline (speedup 1.0000x reference)
import functools

import jax
import jax.numpy as jnp
from jax import lax
from jax.experimental import pallas as pl
from jax.experimental.pallas import tpu as pltpu
from jax.experimental.pallas import tpu_sc as plsc

N = 6_400_000
G = 100_000
G_PAD = 100_352
GSL = G_PAD // 16
NW = 32
EPW = N // NW
B1 = 8_000
CH = 512
ACC = G_PAD + CH
B2 = 10_000
COEF = 0.1 * 0.01

_mesh = plsc.VectorSubcoreMesh(
    core_axis_name="c", subcore_axis_name="s", num_cores=2, num_subcores=16
)
_sc_params = pltpu.CompilerParams(needs_layout_passes=False)


@functools.partial(
    pl.kernel,
    out_type=jax.ShapeDtypeStruct((2, G_PAD), jnp.float32),
    mesh=_mesh,
    scratch_types=[
        pltpu.VMEM((B1,), jnp.float32),
        pltpu.VMEM((B1,), jnp.int32),
        pltpu.VMEM((ACC,), jnp.float32),
        pltpu.VMEM((CH,), jnp.int32),
        pltpu.VMEM((GSL,), jnp.float32),
        pltpu.VMEM_SHARED((G_PAD,), jnp.float32),
    ],
    compiler_params=_sc_params,
)
def _sumsq_kernel(coef_hbm, groups_hbm, out_hbm, cbuf, gbuf, acc, idxbuf, sbuf,
                  acc_sh):
    c = lax.axis_index("c")
    s = lax.axis_index("s")
    wid = s * 2 + c

    zeros = jnp.zeros((16,), jnp.float32)

    @pl.loop(0, GSL // 16)
    def _(i):
        sbuf[pl.ds(i * 16, 16)] = zeros

    pltpu.sync_copy(sbuf, acc_sh.at[pl.ds(s * GSL, GSL)])

    @pl.loop(0, ACC // 16)
    def _(i):
        acc[pl.ds(i * 16, 16)] = zeros

    plsc.subcore_barrier()

    first = jnp.int32(0)
    last = jnp.int32(0)
    for b in range(EPW // B1):
        base = wid * EPW + b * B1
        pltpu.sync_copy(coef_hbm.at[pl.ds(base, B1)], cbuf)
        pltpu.sync_copy(groups_hbm.at[pl.ds(base, B1)], gbuf)
        if b == 0:
            first = jnp.min(gbuf[pl.ds(0, 16)])
        if b == EPW // B1 - 1:
            last = jnp.max(gbuf[pl.ds(B1 - 16, 16)])

        @pl.loop(0, B1 // 16)
        def _(i):
            x = cbuf[pl.ds(i * 16, 16)]
            g = gbuf[pl.ds(i * 16, 16)]
            plsc.addupdate_scatter(acc, [g], x * x)

    first8 = first & ~jnp.int32(7)
    nch = (last - first8) // CH + 1
    lane = jnp.arange(16, dtype=jnp.int32)
    gmax = jnp.full((16,), G_PAD - 1, dtype=jnp.int32)

    @pl.loop(0, nch)
    def _(k):
        cbase = pl.multiple_of(first8 + k * CH, 8)

        @pl.loop(0, CH // 16)
        def _(j):
            idxbuf[pl.ds(j * 16, 16)] = jnp.minimum(cbase + j * 16 + lane, gmax)

        pltpu.sync_copy(acc.at[pl.ds(cbase, CH)], acc_sh.at[idxbuf], add=True)

    plsc.subcore_barrier()
    pltpu.sync_copy(acc_sh.at[pl.ds(s * GSL, GSL)], sbuf)
    pltpu.sync_copy(sbuf, out_hbm.at[c, pl.ds(s * GSL, GSL)])


def _factor_body(p_ref, f_ref):
    sumsq = p_ref[0:1, :] + p_ref[1:2, :]
    norm = jnp.sqrt(sumsq + 1e-12)
    f_ref[...] = jnp.maximum(1.0 - COEF / (norm + 1e-10), 0.0)


_factor_call = pl.pallas_call(
    _factor_body,
    out_shape=jax.ShapeDtypeStruct((1, G_PAD), jnp.float32),
)


@functools.partial(
    pl.kernel,
    out_type=jax.ShapeDtypeStruct((N,), jnp.float32),
    mesh=_mesh,
    scratch_types=[
        pltpu.VMEM((G_PAD,), jnp.float32),
        pltpu.VMEM((B2,), jnp.float32),
        pltpu.VMEM((B2,), jnp.int32),
    ],
    compiler_params=_sc_params,
)
def _apply_kernel(coef_hbm, groups_hbm, factor_hbm, out_hbm, fbuf, cbuf, gbuf):
    c = lax.axis_index("c")
    s = lax.axis_index("s")
    wid = s * 2 + c

    pltpu.sync_copy(factor_hbm, fbuf)

    for b in range(EPW // B2):
        base = wid * EPW + b * B2
        pltpu.sync_copy(coef_hbm.at[pl.ds(base, B2)], cbuf)
        pltpu.sync_copy(groups_hbm.at[pl.ds(base, B2)], gbuf)

        @pl.loop(0, B2 // 16)
        def _(i):
            g = gbuf[pl.ds(i * 16, 16)]
            f = plsc.load_gather(fbuf, [g])
            x = cbuf[pl.ds(i * 16, 16)]
            cbuf[pl.ds(i * 16, 16)] = x * f

        pltpu.sync_copy(cbuf, out_hbm.at[pl.ds(base, B2)])


def kernel(coefficients, groups):
    groups = groups.astype(jnp.int32)
    partials = _sumsq_kernel(coefficients, groups)
    factor = _factor_call(partials).reshape(G_PAD)
    return _apply_kernel(coefficients, groups, factor)


# speedup vs baseline: 1.8169x; 1.8169x over previous
import functools

import jax
import jax.numpy as jnp
from jax import lax
from jax.experimental import pallas as pl
from jax.experimental.pallas import tpu as pltpu
from jax.experimental.pallas import tpu_sc as plsc

N = 6_400_000
G = 100_000
G_PAD = 100_352
GSL = G_PAD // 16
NW = 32
EPW = N // NW
B1 = 10_000
NB1 = EPW // B1
B2 = 4_000
NB2 = EPW // B2
K = 4
K2 = 3
COEF = 0.1 * 0.01

_mesh = plsc.VectorSubcoreMesh(
    core_axis_name="c", subcore_axis_name="s", num_cores=2, num_subcores=16
)
_sc_params = pltpu.CompilerParams(needs_layout_passes=False)


@functools.partial(
    pl.kernel,
    out_type=jax.ShapeDtypeStruct((2, G_PAD), jnp.float32),
    mesh=_mesh,
    scratch_types=[
        [pltpu.VMEM((B1,), jnp.float32) for _ in range(K)],
        [pltpu.VMEM((B1,), jnp.int32) for _ in range(K)],
        pltpu.VMEM((GSL,), jnp.float32),
        pltpu.VMEM_SHARED((G_PAD,), jnp.float32),
        [pltpu.SemaphoreType.DMA for _ in range(K)],
        [pltpu.SemaphoreType.DMA for _ in range(K)],
    ],
    compiler_params=_sc_params,
)
def _sumsq_kernel(coef_hbm, groups_hbm, out_hbm, cbufs, gbufs, sbuf, acc_sh,
                  lsems, ssems):
    c = lax.axis_index("c")
    s = lax.axis_index("s")
    wid = s * 2 + c

    zeros = jnp.zeros((16,), jnp.float32)

    @pl.loop(0, GSL // 16)
    def _(i):
        sbuf[pl.ds(i * 16, 16)] = zeros

    pltpu.sync_copy(sbuf, acc_sh.at[pl.ds(s * GSL, GSL)])
    plsc.subcore_barrier()

    def issue_loads(b):
        p = b % K
        base = wid * EPW + b * B1
        return (
            pltpu.async_copy(coef_hbm.at[pl.ds(base, B1)], cbufs[p], lsems[p]),
            pltpu.async_copy(groups_hbm.at[pl.ds(base, B1)], gbufs[p], lsems[p]),
        )

    loads = {b: issue_loads(b) for b in range(min(2, NB1))}
    scats = {}
    for b in range(NB1):
        p = b % K
        if b - 2 in scats:
            scats[b - 2].wait()
        if b + 2 < NB1:
            loads[b + 2] = issue_loads(b + 2)
        for h in loads.pop(b):
            h.wait()

        @pl.loop(0, B1 // 16)
        def _(i):
            x = cbufs[p][pl.ds(i * 16, 16)]
            cbufs[p][pl.ds(i * 16, 16)] = x * x

        scats[b] = pltpu.async_copy(
            cbufs[p], acc_sh.at[gbufs[p]], ssems[p], add=True
        )

    for b in (NB1 - 2, NB1 - 1):
        if b in scats:
            scats[b].wait()

    plsc.subcore_barrier()
    pltpu.sync_copy(acc_sh.at[pl.ds(s * GSL, GSL)], sbuf)
    pltpu.sync_copy(sbuf, out_hbm.at[c, pl.ds(s * GSL, GSL)])


def _factor_body(p_ref, f_ref):
    sumsq = p_ref[0:1, :] + p_ref[1:2, :]
    norm = jnp.sqrt(sumsq + 1e-12)
    f_ref[...] = jnp.maximum(1.0 - COEF / (norm + 1e-10), 0.0)


_factor_call = pl.pallas_call(
    _factor_body,
    out_shape=jax.ShapeDtypeStruct((1, G_PAD), jnp.float32),
)


@functools.partial(
    pl.kernel,
    out_type=jax.ShapeDtypeStruct((N,), jnp.float32),
    mesh=_mesh,
    scratch_types=[
        pltpu.VMEM((G_PAD,), jnp.float32),
        [pltpu.VMEM((B2,), jnp.float32) for _ in range(K2)],
        [pltpu.VMEM((B2,), jnp.int32) for _ in range(K2)],
        [pltpu.SemaphoreType.DMA for _ in range(K2)],
        [pltpu.SemaphoreType.DMA for _ in range(K2)],
    ],
    compiler_params=_sc_params,
)
def _apply_kernel(coef_hbm, groups_hbm, factor_hbm, out_hbm, fbuf, cbufs,
                  gbufs, lsems, osems):
    c = lax.axis_index("c")
    s = lax.axis_index("s")
    wid = s * 2 + c

    pltpu.sync_copy(factor_hbm, fbuf)

    def issue_loads(b):
        p = b % K2
        base = wid * EPW + b * B2
        return (
            pltpu.async_copy(coef_hbm.at[pl.ds(base, B2)], cbufs[p], lsems[p]),
            pltpu.async_copy(groups_hbm.at[pl.ds(base, B2)], gbufs[p], lsems[p]),
        )

    loads = {0: issue_loads(0)}
    stores = {}
    for b in range(NB2):
        p = b % K2
        if b - 2 in stores:
            stores[b - 2].wait()
        if b + 1 < NB2:
            loads[b + 1] = issue_loads(b + 1)
        for h in loads.pop(b):
            h.wait()

        @pl.loop(0, B2 // 16)
        def _(i):
            g = gbufs[p][pl.ds(i * 16, 16)]
            f = plsc.load_gather(fbuf, [g])
            x = cbufs[p][pl.ds(i * 16, 16)]
            cbufs[p][pl.ds(i * 16, 16)] = x * f

        base_o = wid * EPW + b * B2
        stores[b] = pltpu.async_copy(
            cbufs[p], out_hbm.at[pl.ds(base_o, B2)], osems[p]
        )

    for b in (NB2 - 2, NB2 - 1):
        if b in stores:
            stores[b].wait()


def kernel(coefficients, groups):
    groups = groups.astype(jnp.int32)
    partials = _sumsq_kernel(coefficients, groups)
    factor = _factor_call(partials).reshape(G_PAD)
    return _apply_kernel(coefficients, groups, factor)
